# baseline (speedup 1.0000x reference)
import functools

import jax
import jax.numpy as jnp
from jax import lax
from jax.experimental import pallas as pl
from jax.experimental.pallas import tpu as pltpu
from jax.experimental.pallas import tpu_sc as plsc

B = 16384
D = 128

_info = plsc.get_sparse_core_info()
NC = _info.num_cores
NS = _info.num_subcores
L = _info.num_lanes
NW = NC * NS
BPW = B // NW
CHUNK = 128
NCHUNK = BPW // CHUNK
GROUPS = CHUNK // L


def _skipgram_body(center_hbm, context_hbm, emb_hbm, oemb_hbm, out_hbm,
                   idx_c, idx_x, rc0, rc1, rx0, rx1, scores,
                   s0, s1, s2, s3):
    wid = lax.axis_index("s") * NC + lax.axis_index("c")
    base = wid * BPW

    pltpu.sync_copy(center_hbm.at[pl.ds(base, BPW)], idx_c)
    pltpu.sync_copy(context_hbm.at[pl.ds(base, BPW)], idx_x)

    rc = (rc0, rc1)
    rx = (rx0, rx1)
    sem_c = (s0, s1)
    sem_x = (s2, s3)

    def start(c):
        b = c & 1
        hc = pltpu.async_copy(
            emb_hbm.at[idx_c.at[pl.ds(c * CHUNK, CHUNK)]], rc[b], sem_c[b])
        hx = pltpu.async_copy(
            oemb_hbm.at[idx_x.at[pl.ds(c * CHUNK, CHUNK)]], rx[b], sem_x[b])
        return hc, hx

    pending = start(0)
    for c in range(NCHUNK):
        b = c & 1
        nxt = start(c + 1) if c + 1 < NCHUNK else None
        pending[0].wait()
        pending[1].wait()
        rcb = rc[b]
        rxb = rx[b]

        @plsc.parallel_loop(0, GROUPS, unroll=2)
        def _group(g, rcb=rcb, rxb=rxb, c=c):
            lane = lax.iota(jnp.int32, L)
            row0 = pl.multiple_of(g * L, L)
            accs = []
            for i in range(L):
                r = row0 + i
                a = rcb[r, pl.ds(0, L)] * rxb[r, pl.ds(0, L)]
                for d in range(1, D // L):
                    a = a + rcb[r, pl.ds(d * L, L)] * rxb[r, pl.ds(d * L, L)]
                accs.append(a)
            s = 1
            while len(accs) > 1:
                perm = lane ^ s
                keep = (lane & s) == 0
                nxt_accs = []
                for k in range(0, len(accs), 2):
                    a, bb = accs[k], accs[k + 1]
                    sa = a.at[perm].get(mode="promise_in_bounds")
                    sb = bb.at[perm].get(mode="promise_in_bounds")
                    nxt_accs.append(jnp.where(keep, a, bb) +
                                    jnp.where(keep, sa, sb))
                accs = nxt_accs
                s *= 2
            scores[pl.ds(c * CHUNK + row0, L)] = accs[0]

        pending = nxt

    pltpu.sync_copy(scores, out_hbm.at[pl.ds(base, BPW)])


_skipgram = functools.partial(
    pl.kernel,
    out_type=jax.ShapeDtypeStruct((B,), jnp.float32),
    mesh=plsc.VectorSubcoreMesh(core_axis_name="c", subcore_axis_name="s"),
    scratch_types=[
        pltpu.VMEM((BPW,), jnp.int32),
        pltpu.VMEM((BPW,), jnp.int32),
        pltpu.VMEM((CHUNK, D), jnp.float32),
        pltpu.VMEM((CHUNK, D), jnp.float32),
        pltpu.VMEM((CHUNK, D), jnp.float32),
        pltpu.VMEM((CHUNK, D), jnp.float32),
        pltpu.VMEM((BPW,), jnp.float32),
        pltpu.SemaphoreType.DMA,
        pltpu.SemaphoreType.DMA,
        pltpu.SemaphoreType.DMA,
        pltpu.SemaphoreType.DMA,
    ],
)(_skipgram_body)


def kernel(center_nodes, context_nodes, embeddings, output_embeddings):
    return _skipgram(center_nodes, context_nodes, embeddings,
                     output_embeddings)


# speedup vs baseline: 1.3046x; 1.3046x over previous
import functools

import jax
import jax.numpy as jnp
from jax import lax
from jax.experimental import pallas as pl
from jax.experimental.pallas import tpu as pltpu
from jax.experimental.pallas import tpu_sc as plsc

B = 16384
D = 128

_info = plsc.get_sparse_core_info()
NC = _info.num_cores
NS = _info.num_subcores
L = _info.num_lanes
NW = NC * NS
BPW = B // NW
CHUNK = 128
NCHUNK = BPW // CHUNK
GROUPS = CHUNK // L


def _skipgram_body(center_hbm, context_hbm, emb_hbm, oemb_hbm, out_hbm,
                   idx_c, idx_x, rc0, rc1, rx0, rx1, scores,
                   s0, s1, s2, s3):
    wid = lax.axis_index("s") * NC + lax.axis_index("c")
    base = wid * BPW

    pltpu.sync_copy(center_hbm.at[pl.ds(base, BPW)], idx_c)
    pltpu.sync_copy(context_hbm.at[pl.ds(base, BPW)], idx_x)

    rc = (rc0, rc1)
    rx = (rx0, rx1)
    sem_c = (s0, s1)
    sem_x = (s2, s3)

    def start(c):
        b = c & 1
        hc = pltpu.async_copy(
            emb_hbm.at[idx_c.at[pl.ds(c * CHUNK, CHUNK)]], rc[b], sem_c[b])
        hx = pltpu.async_copy(
            oemb_hbm.at[idx_x.at[pl.ds(c * CHUNK, CHUNK)]], rx[b], sem_x[b])
        return hc, hx

    pending = start(0)
    for c in range(NCHUNK):
        b = c & 1
        nxt = start(c + 1) if c + 1 < NCHUNK else None
        pending[0].wait()
        pending[1].wait()
        rcb = rc[b]
        rxb = rx[b]

        def _group(g, carry, rcb=rcb, rxb=rxb, c=c):
            lane = lax.iota(jnp.int32, L)
            perms = [lane ^ s for s in (1, 2, 4, 8)]
            keeps = [(lane & s) == 0 for s in (1, 2, 4, 8)]
            row0 = pl.multiple_of(g * L, L)

            def merge(a, bb, lvl):
                sa = a.at[perms[lvl]].get(mode="promise_in_bounds")
                sb = bb.at[perms[lvl]].get(mode="promise_in_bounds")
                return (jnp.where(keeps[lvl], a, bb) +
                        jnp.where(keeps[lvl], sa, sb))

            pend = [None, None, None, None]
            for i in range(L):
                r = row0 + i
                a = rcb[r, pl.ds(0, L)] * rxb[r, pl.ds(0, L)]
                for d in range(1, D // L):
                    a = a + rcb[r, pl.ds(d * L, L)] * rxb[r, pl.ds(d * L, L)]
                lvl = 0
                while lvl < 4 and pend[lvl] is not None:
                    a = merge(pend[lvl], a, lvl)
                    pend[lvl] = None
                    lvl += 1
                if lvl < 4:
                    pend[lvl] = a
            scores[pl.ds(c * CHUNK + row0, L)] = a
            return carry

        lax.fori_loop(0, GROUPS, _group, 0)
        pending = nxt

    pltpu.sync_copy(scores, out_hbm.at[pl.ds(base, BPW)])


_skipgram = functools.partial(
    pl.kernel,
    out_type=jax.ShapeDtypeStruct((B,), jnp.float32),
    mesh=plsc.VectorSubcoreMesh(core_axis_name="c", subcore_axis_name="s"),
    scratch_types=[
        pltpu.VMEM((BPW,), jnp.int32),
        pltpu.VMEM((BPW,), jnp.int32),
        pltpu.VMEM((CHUNK, D), jnp.float32),
        pltpu.VMEM((CHUNK, D), jnp.float32),
        pltpu.VMEM((CHUNK, D), jnp.float32),
        pltpu.VMEM((CHUNK, D), jnp.float32),
        pltpu.VMEM((BPW,), jnp.float32),
        pltpu.SemaphoreType.DMA,
        pltpu.SemaphoreType.DMA,
        pltpu.SemaphoreType.DMA,
        pltpu.SemaphoreType.DMA,
    ],
)(_skipgram_body)


def kernel(center_nodes, context_nodes, embeddings, output_embeddings):
    return _skipgram(center_nodes, context_nodes, embeddings,
                     output_embeddings)


# speedup vs baseline: 1.3470x; 1.0325x over previous
import functools

import jax
import jax.numpy as jnp
from jax import lax
from jax.experimental import pallas as pl
from jax.experimental.pallas import tpu as pltpu
from jax.experimental.pallas import tpu_sc as plsc

B = 16384
D = 128

_info = plsc.get_sparse_core_info()
NC = _info.num_cores
NS = _info.num_subcores
L = _info.num_lanes
NW = NC * NS
BPW = B // NW
CHUNK = 128
NCHUNK = BPW // CHUNK
GROUPS = CHUNK // L


def _skipgram_body(center_hbm, context_hbm, emb_hbm, oemb_hbm, out_hbm,
                   idx_c, idx_x, rc0, rc1, rx0, rx1, scores,
                   s0, s1, s2, s3):
    wid = lax.axis_index("s") * NC + lax.axis_index("c")
    base = wid * BPW

    hic = pltpu.async_copy(center_hbm.at[pl.ds(base, BPW)], idx_c, s0)
    hix = pltpu.async_copy(context_hbm.at[pl.ds(base, BPW)], idx_x, s1)
    hic.wait()
    hix.wait()

    rc = (rc0, rc1)
    rx = (rx0, rx1)
    sem_c = (s0, s1)
    sem_x = (s2, s3)

    def start(c):
        b = c & 1
        hc = pltpu.async_copy(
            emb_hbm.at[idx_c.at[pl.ds(c * CHUNK, CHUNK)]], rc[b], sem_c[b])
        hx = pltpu.async_copy(
            oemb_hbm.at[idx_x.at[pl.ds(c * CHUNK, CHUNK)]], rx[b], sem_x[b])
        return hc, hx

    pending = start(0)
    for c in range(NCHUNK):
        b = c & 1
        nxt = start(c + 1) if c + 1 < NCHUNK else None
        pending[0].wait()
        pending[1].wait()
        rcb = rc[b]
        rxb = rx[b]

        def _group(g, carry, rcb=rcb, rxb=rxb, c=c):
            lane = lax.iota(jnp.int32, L)
            perms = [lane ^ s for s in (1, 2, 4, 8)]
            keeps = [(lane & s) == 0 for s in (1, 2, 4, 8)]
            row0 = pl.multiple_of(g * L, L)

            def merge(a, bb, lvl):
                m1 = jnp.where(keeps[lvl], a, bb)
                m2 = jnp.where(keeps[lvl], bb, a)
                return m1 + m2.at[perms[lvl]].get(mode="promise_in_bounds")

            pend = [None, None, None, None]
            for i in range(L):
                r = row0 + i
                a = rcb[r, pl.ds(0, L)] * rxb[r, pl.ds(0, L)]
                for d in range(1, D // L):
                    a = a + rcb[r, pl.ds(d * L, L)] * rxb[r, pl.ds(d * L, L)]
                lvl = 0
                while lvl < 4 and pend[lvl] is not None:
                    a = merge(pend[lvl], a, lvl)
                    pend[lvl] = None
                    lvl += 1
                if lvl < 4:
                    pend[lvl] = a
            scores[pl.ds(c * CHUNK + row0, L)] = a
            return carry

        lax.fori_loop(0, GROUPS, _group, 0)
        pending = nxt

    pltpu.sync_copy(scores, out_hbm.at[pl.ds(base, BPW)])


_skipgram = functools.partial(
    pl.kernel,
    out_type=jax.ShapeDtypeStruct((B,), jnp.float32),
    mesh=plsc.VectorSubcoreMesh(core_axis_name="c", subcore_axis_name="s"),
    scratch_types=[
        pltpu.VMEM((BPW,), jnp.int32),
        pltpu.VMEM((BPW,), jnp.int32),
        pltpu.VMEM((CHUNK, D), jnp.float32),
        pltpu.VMEM((CHUNK, D), jnp.float32),
        pltpu.VMEM((CHUNK, D), jnp.float32),
        pltpu.VMEM((CHUNK, D), jnp.float32),
        pltpu.VMEM((BPW,), jnp.float32),
        pltpu.SemaphoreType.DMA,
        pltpu.SemaphoreType.DMA,
        pltpu.SemaphoreType.DMA,
        pltpu.SemaphoreType.DMA,
    ],
)(_skipgram_body)


def kernel(center_nodes, context_nodes, embeddings, output_embeddings):
    return _skipgram(center_nodes, context_nodes, embeddings,
                     output_embeddings)
